# baseline (speedup 1.0000x reference)
import jax
import jax.numpy as jnp
from jax import lax
from jax.experimental import pallas as pl

B, T, D = 16, 2048, 768
N = 800
MAX_BBOX = 100


TCHUNK = 512
NT = T // TCHUNK


def _body(x_ref, ids_col_ref, ids_row_ref, feat_ref, w1_ref, b1_ref,
          w2_ref, b2_ref, pw_ref, pb_ref, vis_ref, mask_ref, retx_ref):
    b = pl.program_id(0)
    t = pl.program_id(1)
    x = x_ref[...]
    s = jnp.sum(x, axis=1, keepdims=True) * (1.0 / T)

    @pl.when(t == 0)
    def _init():
        retx_ref[...] = s

    @pl.when(t > 0)
    def _acc():
        retx_ref[...] += s

    @pl.when(jnp.logical_and(b == 0, t == 0))
    def _boxes():
        ids_col = ids_col_ref[...]
        ids_row = ids_row_ref[...]
        f = feat_ref[...]
        h = jnp.maximum(
            jnp.dot(f, w1_ref[...], preferred_element_type=jnp.float32)
            + b1_ref[...], 0.0)
        f2 = (jnp.dot(h, w2_ref[...], preferred_element_type=jnp.float32)
              + b2_ref[...])
        xp = (jnp.dot(f2, pw_ref[...], preferred_element_type=jnp.float32)
              + pb_ref[...])

        eq = (ids_col == ids_row)
        ri = lax.broadcasted_iota(jnp.int32, (N, N), 0)
        ci = lax.broadcasted_iota(jnp.int32, (N, N), 1)
        before = jnp.logical_and(eq, ri < ci).astype(jnp.int32)
        slot_row = jnp.sum(before, axis=0, keepdims=True)
        pos_row = ids_row * MAX_BBOX + slot_row

        pp = lax.broadcasted_iota(jnp.int32, (B * MAX_BBOX, N), 0)
        P = (pp == pos_row).astype(jnp.float32)
        vis_ref[...] = jnp.dot(P, xp, preferred_element_type=jnp.float32)

        img = lax.broadcasted_iota(jnp.int32, (B, N), 0)
        counts = jnp.sum((img == ids_row).astype(jnp.int32), axis=1,
                         keepdims=True)
        jj = lax.broadcasted_iota(jnp.int32, (B, MAX_BBOX), 1)
        mask_ref[...] = (jj < counts).astype(jnp.float32)


def kernel(inputs, bboxes, features, mlp_w1, mlp_b1, mlp_w2, mlp_b2,
           tr_w1, tr_b1, tr_w2, tr_b2, proj_w, proj_b):
    del tr_w1, tr_b1, tr_w2, tr_b2
    ids = bboxes[:, 0]
    ids_col = ids.reshape(N, 1)
    ids_row = ids.reshape(1, N)
    feat = features[:, 1:]
    pw = proj_w[768:]

    vis_flat, att_mask, ret_x = pl.pallas_call(
        _body,
        grid=(B, NT),
        in_specs=[
            pl.BlockSpec((1, TCHUNK, D), lambda b, t: (b, t, 0)),
            pl.BlockSpec((N, 1), lambda b, t: (0, 0)),
            pl.BlockSpec((1, N), lambda b, t: (0, 0)),
            pl.BlockSpec((N, 256), lambda b, t: (0, 0)),
            pl.BlockSpec((256, D), lambda b, t: (0, 0)),
            pl.BlockSpec((1, D), lambda b, t: (0, 0)),
            pl.BlockSpec((D, D), lambda b, t: (0, 0)),
            pl.BlockSpec((1, D), lambda b, t: (0, 0)),
            pl.BlockSpec((D, D), lambda b, t: (0, 0)),
            pl.BlockSpec((1, D), lambda b, t: (0, 0)),
        ],
        out_specs=[
            pl.BlockSpec((B * MAX_BBOX, D), lambda b, t: (0, 0)),
            pl.BlockSpec((B, MAX_BBOX), lambda b, t: (0, 0)),
            pl.BlockSpec((1, 1, D), lambda b, t: (b, 0, 0)),
        ],
        out_shape=[
            jax.ShapeDtypeStruct((B * MAX_BBOX, D), jnp.float32),
            jax.ShapeDtypeStruct((B, MAX_BBOX), jnp.float32),
            jax.ShapeDtypeStruct((B, 1, D), jnp.float32),
        ],
    )(inputs, ids_col, ids_row, feat, mlp_w1, mlp_b1.reshape(1, D),
      mlp_w2, mlp_b2.reshape(1, D), pw, proj_b.reshape(1, D))

    return (vis_flat.reshape(B, MAX_BBOX, D), att_mask, ret_x.reshape(B, D))


# speedup vs baseline: 1.4267x; 1.4267x over previous
import jax
import jax.numpy as jnp
from jax import lax
from jax.experimental import pallas as pl

B, T, D = 16, 2048, 768
N = 800
MAX_BBOX = 100


BCHUNK = 2
NB = B // BCHUNK


def _body(x_ref, ids_col_ref, ids_row_ref, feat_ref, w1_ref, b1_ref,
          w2_ref, b2_ref, pw_ref, pb_ref, vis_ref, mask_ref, retx_ref):
    b = pl.program_id(0)
    x = x_ref[...]
    retx_ref[...] = jnp.sum(x, axis=1, keepdims=True) * (1.0 / T)

    @pl.when(b == 0)
    def _boxes():
        ids_col = ids_col_ref[...]
        ids_row = ids_row_ref[...]
        f = feat_ref[...]
        h = jnp.maximum(
            jnp.dot(f, w1_ref[...], preferred_element_type=jnp.float32)
            + b1_ref[...], 0.0)
        f2 = (jnp.dot(h, w2_ref[...], preferred_element_type=jnp.float32)
              + b2_ref[...])
        xp = (jnp.dot(f2, pw_ref[...], preferred_element_type=jnp.float32)
              + pb_ref[...])

        eq = (ids_col == ids_row)
        ri = lax.broadcasted_iota(jnp.int32, (N, N), 0)
        ci = lax.broadcasted_iota(jnp.int32, (N, N), 1)
        before = jnp.logical_and(eq, ri < ci).astype(jnp.int32)
        slot_row = jnp.sum(before, axis=0, keepdims=True)
        pos_row = ids_row * MAX_BBOX + slot_row

        pp = lax.broadcasted_iota(jnp.int32, (B * MAX_BBOX, N), 0)
        P = (pp == pos_row).astype(jnp.float32)
        vis_ref[...] = jnp.dot(P, xp, preferred_element_type=jnp.float32)

        img = lax.broadcasted_iota(jnp.int32, (B, N), 0)
        counts = jnp.sum((img == ids_row).astype(jnp.int32), axis=1,
                         keepdims=True)
        jj = lax.broadcasted_iota(jnp.int32, (B, MAX_BBOX), 1)
        mask_ref[...] = (jj < counts).astype(jnp.float32)


def kernel(inputs, bboxes, features, mlp_w1, mlp_b1, mlp_w2, mlp_b2,
           tr_w1, tr_b1, tr_w2, tr_b2, proj_w, proj_b):
    del tr_w1, tr_b1, tr_w2, tr_b2
    ids = bboxes[:, 0]
    ids_col = ids.reshape(N, 1)
    ids_row = ids.reshape(1, N)
    feat = features[:, 1:]
    pw = proj_w[768:]

    vis_flat, att_mask, ret_x = pl.pallas_call(
        _body,
        grid=(NB,),
        in_specs=[
            pl.BlockSpec((BCHUNK, T, D), lambda b: (b, 0, 0)),
            pl.BlockSpec((N, 1), lambda b: (0, 0)),
            pl.BlockSpec((1, N), lambda b: (0, 0)),
            pl.BlockSpec((N, 256), lambda b: (0, 0)),
            pl.BlockSpec((256, D), lambda b: (0, 0)),
            pl.BlockSpec((1, D), lambda b: (0, 0)),
            pl.BlockSpec((D, D), lambda b: (0, 0)),
            pl.BlockSpec((1, D), lambda b: (0, 0)),
            pl.BlockSpec((D, D), lambda b: (0, 0)),
            pl.BlockSpec((1, D), lambda b: (0, 0)),
        ],
        out_specs=[
            pl.BlockSpec((B * MAX_BBOX, D), lambda b: (0, 0)),
            pl.BlockSpec((B, MAX_BBOX), lambda b: (0, 0)),
            pl.BlockSpec((BCHUNK, 1, D), lambda b: (b, 0, 0)),
        ],
        out_shape=[
            jax.ShapeDtypeStruct((B * MAX_BBOX, D), jnp.float32),
            jax.ShapeDtypeStruct((B, MAX_BBOX), jnp.float32),
            jax.ShapeDtypeStruct((B, 1, D), jnp.float32),
        ],
    )(inputs, ids_col, ids_row, feat, mlp_w1, mlp_b1.reshape(1, D),
      mlp_w2, mlp_b2.reshape(1, D), pw, proj_b.reshape(1, D))

    return (vis_flat.reshape(B, MAX_BBOX, D), att_mask, ret_x.reshape(B, D))
